# baseline (speedup 1.0000x reference)
import functools

import jax
import jax.numpy as jnp
from jax import lax
from jax.experimental import pallas as pl
from jax.experimental.pallas import tpu as pltpu
from jax.experimental.pallas import tpu_sc as plsc

F = 4
NPTS = 65536
NXG = 81
NYG = 81
NCELL = NXG * NYG
CPAD = 6656
NPLANE = 5
ACCW = NPLANE * CPAD
Q = 512
NSUB = 16
NCORE = 2
GROUP = 8
PPT = NPTS // GROUP
VPT = PPT // 16
NVC = CPAD // 16
X_MIN = -4.0
SX = 0.1
GMAX = NXG - 1


def _sc_body(pts_hbm, msk_hbm, out_hbm, pts_v, msk_v, acc_v, red_v, red2_v,
             hist_v, nge_v, bigkey_v, eqidx_v, sel_v, val_v, wb_v, rankf_v,
             feat8_v, sh_part, sh_aux, sh_acc):
    c = lax.axis_index("c")
    s = lax.axis_index("s")
    fc = s // GROUP
    f = c * 2 + fc
    g = s % GROUP
    p0 = g * PPT
    lanes = lax.iota(jnp.int32, 16)
    zf16 = jnp.zeros((16,), jnp.float32)
    one16 = jnp.ones((16,), jnp.float32)

    pltpu.sync_copy(pts_hbm.at[pl.ds((f * NPTS + p0) * 4, PPT * 4)], pts_v)
    pltpu.sync_copy(msk_hbm.at[pl.ds(f * NPTS + p0, PPT)], msk_v)

    def zero_body(i, _):
        for u in range(8):
            acc_v[pl.ds(i * 128 + u * 16, 16)] = zf16
        return 0
    lax.fori_loop(0, ACCW // 128, zero_body, 0)

    def bin_body(v, _):
      for u in range(2):
        base = v * 32 + u * 16
        pidx = base * 4 + lanes * 4
        px = plsc.load_gather(pts_v, [pidx])
        py = plsc.load_gather(pts_v, [pidx + 1])
        pz = plsc.load_gather(pts_v, [pidx + 2])
        pi = plsc.load_gather(pts_v, [pidx + 3])
        m = msk_v[pl.ds(base, 16)]
        valid = ((m != 0.0)
                 & (px >= -4.0) & (px <= 4.0)
                 & (py >= -4.0) & (py <= 4.0)
                 & (pz >= -4.0) & (pz <= 4.0))
        gx = jnp.clip(((px - X_MIN) / SX).astype(jnp.int32), 0, GMAX)
        gy = jnp.clip(((py - X_MIN) / SX).astype(jnp.int32), 0, GMAX)
        cell = gx * NYG + gy
        plsc.addupdate_scatter(acc_v, [cell], one16, mask=valid)
        plsc.addupdate_scatter(acc_v, [cell + CPAD], px, mask=valid)
        plsc.addupdate_scatter(acc_v, [cell + 2 * CPAD], py, mask=valid)
        plsc.addupdate_scatter(acc_v, [cell + 3 * CPAD], pz, mask=valid)
        plsc.addupdate_scatter(acc_v, [cell + 4 * CPAD], pi, mask=valid)
      return 0
    lax.fori_loop(0, VPT // 2, bin_body, 0)

    half = ACCW // 2
    sliw = half // GROUP
    off = g * sliw
    gbase = fc * GROUP * half
    for r in range(2):
        pltpu.sync_copy(acc_v.at[pl.ds(r * half, half)],
                        sh_part.at[pl.ds(s * half, half)])
        plsc.subcore_barrier()
        pltpu.sync_copy(sh_part.at[pl.ds(gbase + off, sliw)], red2_v)
        for t in range(1, GROUP):
            pltpu.sync_copy(sh_part.at[pl.ds(gbase + t * half + off, sliw)],
                            red_v)

            def add_body(i, _):
                for u in range(5):
                    o = i * 80 + u * 16
                    red2_v[pl.ds(o, 16)] = (red2_v[pl.ds(o, 16)]
                                            + red_v[pl.ds(o, 16)])
                return 0
            lax.fori_loop(0, sliw // 80, add_body, 0)
        pltpu.sync_copy(red2_v,
                        sh_acc.at[pl.ds(fc * ACCW + r * half + off, sliw)])
        plsc.subcore_barrier()

    @pl.when(g == 0)
    def _topk():
        pltpu.sync_copy(sh_acc.at[pl.ds(fc * ACCW, ACCW)], acc_v)

        for i in range(9):
            hist_v[pl.ds(i * 16, 16)] = jnp.zeros((16,), jnp.int32)
        for i in range(32):
            bigkey_v[pl.ds(i * 16, 16)] = jnp.full((16,), -1, jnp.int32)
        for r in range(5, 8):
            def z8_body(i, _, r=r):
                feat8_v[pl.ds(r * Q + i * 16, 16)] = zf16
                return 0
            lax.fori_loop(0, Q // 16, z8_body, 0)

        def hist_body(v, _):
            for u in range(2):
                cnt = acc_v[pl.ds(v * 32 + u * 16, 16)]
                ci = cnt.astype(jnp.int32)
                bin_ = jnp.minimum(ci, 128)
                cellidx = v * 32 + u * 16 + lanes
                plsc.addupdate_scatter(hist_v, [bin_],
                                       jnp.ones((16,), jnp.int32),
                                       mask=cellidx < NCELL)
            return 0
        lax.fori_loop(0, NVC // 2, hist_body, 0)

        carry = jnp.zeros((16,), jnp.int32)
        for vi in range(8, -1, -1):
            h = hist_v[pl.ds(vi * 16, 16)]
            cs = lax.rev(plsc.cumsum(lax.rev(h, (0,))), (0,))
            nge_v[pl.ds(vi * 16, 16)] = cs + carry
            carry = carry + jnp.full((16,), jnp.sum(h))

        T = jnp.int32(-1)
        for vi in range(9):
            tvec = lanes + vi * 16
            ngev = nge_v[pl.ds(vi * 16, 16)]
            T = jnp.maximum(T, jnp.max(jnp.where(ngev >= Q, tvec, -1)))
        M = jnp.int32(0)
        for vi in range(9):
            tvec = lanes + vi * 16
            ngev = nge_v[pl.ds(vi * 16, 16)]
            M = jnp.maximum(M, jnp.max(jnp.where(tvec == T + 1, ngev, 0)))

        def comp_body(v, bases):
            bigbase, eqbase = bases
            for u in range(2):
                ci = acc_v[pl.ds(v * 32 + u * 16, 16)].astype(jnp.int32)
                cellidx = v * 32 + u * 16 + lanes
                iscell = cellidx < NCELL
                big = (ci > T) & iscell
                eq = (ci == T) & iscell
                key = ci * 8192 + (8191 - cellidx)
                bpos = bigbase + plsc.cumsum(big.astype(jnp.int32)) - 1
                plsc.store_scatter(bigkey_v, [bpos], key, mask=big)
                epos = eqbase + plsc.cumsum(eq.astype(jnp.int32)) - 1
                plsc.store_scatter(eqidx_v, [epos], cellidx, mask=eq)
                nb = plsc.all_reduce_population_count(big).astype(jnp.int32)
                ne = plsc.all_reduce_population_count(eq).astype(jnp.int32)
                bigbase = bigbase + nb
                eqbase = eqbase + ne
            return (bigbase, eqbase)
        lax.fori_loop(0, NVC // 2, comp_body,
                      (jnp.zeros((16,), jnp.int32), jnp.zeros((16,), jnp.int32)))

        pltpu.sync_copy(bigkey_v, sh_aux.at[pl.ds(fc * 2048, Q)])
        rankf_v[pl.ds(0, 16)] = jnp.full((16,), M)
        rankf_v[pl.ds(16, 16)] = jnp.full((16,), T)
        pltpu.sync_copy(rankf_v.at[pl.ds(0, 32)],
                        sh_aux.at[pl.ds(fc * 2048 + 512, 32)])

    plsc.subcore_barrier()
    pltpu.sync_copy(sh_aux.at[pl.ds(fc * 2048, Q)], bigkey_v)

    def wb_body(j, _):
        wv = bigkey_v[pl.ds(j * 16, 16)]
        for l in range(16):
            wl = jnp.max(jnp.where(lanes == l, wv, jnp.int32(-2**31 + 1)))
            wb_v[pl.ds((j * 16 + l) * 16, 16)] = jnp.full((16,), wl)
        return 0
    lax.fori_loop(0, 32, wb_body, 0)

    for u in range(4):
        kv = bigkey_v[pl.ds((g * 4 + u) * 16, 16)]

        def cnt_body(j8, r, kv=kv):
            for u8 in range(8):
                r = r + (wb_v[pl.ds(j8 * 128 + u8 * 16, 16)] > kv
                         ).astype(jnp.int32)
            return r
        rankv = lax.fori_loop(0, Q // 8, cnt_body, jnp.zeros((16,), jnp.int32))
        rankf_v[pl.ds(u * 16, 16)] = rankv
    pltpu.sync_copy(rankf_v, sh_aux.at[pl.ds(fc * 2048 + 1024 + g * 64, 64)])
    plsc.subcore_barrier()

    @pl.when(g == 0)
    def _emit():
        pltpu.sync_copy(sh_aux.at[pl.ds(fc * 2048 + 1024, Q)],
                        wb_v.at[pl.ds(0, Q)])
        pltpu.sync_copy(sh_aux.at[pl.ds(fc * 2048 + 512, 32)],
                        rankf_v.at[pl.ds(0, 32)])
        M = jnp.max(rankf_v[pl.ds(0, 16)])
        T = jnp.max(rankf_v[pl.ds(16, 16)])

        def scat_body(i, _):
            kv = bigkey_v[pl.ds(i * 16, 16)]
            rankv = wb_v[pl.ds(i * 16, 16)]
            mb = (i * 16 + lanes) < M
            idx = 8191 - (kv & 8191)
            valf = (kv >> 13).astype(jnp.float32)
            plsc.store_scatter(sel_v, [rankv], idx, mask=mb)
            plsc.store_scatter(val_v, [rankv], valf, mask=mb)
            return 0
        lax.fori_loop(0, 32, scat_body, 0)

        def eq_body(v, _):
            t = v * 16 + lanes
            me = t < (Q - M)
            eidx = eqidx_v[pl.ds(v * 16, 16)]
            plsc.store_scatter(sel_v, [M + t], eidx, mask=me)
            plsc.store_scatter(val_v, [M + t],
                              jnp.full((16,), T.astype(jnp.float32)), mask=me)
            return 0
        lax.fori_loop(0, Q // 16, eq_body, 0)

        def gat_body(v, _):
            base = v * 16
            idxv = sel_v[pl.ds(base, 16)]
            valv = val_v[pl.ds(base, 16)]
            denom = jnp.maximum(valv, 1.0)
            for r in range(4):
                sr = plsc.load_gather(acc_v, [idxv + (r + 1) * CPAD])
                feat8_v[pl.ds(r * Q + base, 16)] = sr / denom
            feat8_v[pl.ds(4 * Q + base, 16)] = valv
            return 0
        lax.fori_loop(0, Q // 16, gat_body, 0)

        pltpu.sync_copy(feat8_v, out_hbm.at[pl.ds(f * 8 * Q, 8 * Q)])


@functools.lru_cache(maxsize=1)
def _sc_encode():
    mesh = plsc.VectorSubcoreMesh(core_axis_name="c", subcore_axis_name="s",
                                  num_cores=NCORE, num_subcores=NSUB)
    return pl.kernel(
        _sc_body,
        out_type=jax.ShapeDtypeStruct((F * 8 * Q,), jnp.float32),
        mesh=mesh,
        compiler_params=pltpu.CompilerParams(needs_layout_passes=False),
        scratch_types=[
            pltpu.VMEM((4 * PPT,), jnp.float32),
            pltpu.VMEM((PPT,), jnp.float32),
            pltpu.VMEM((ACCW,), jnp.float32),
            pltpu.VMEM((ACCW // 2 // GROUP,), jnp.float32),
            pltpu.VMEM((ACCW // 2 // GROUP,), jnp.float32),
            pltpu.VMEM((144,), jnp.int32),
            pltpu.VMEM((144,), jnp.int32),
            pltpu.VMEM((Q,), jnp.int32),
            pltpu.VMEM((CPAD,), jnp.int32),
            pltpu.VMEM((Q,), jnp.int32),
            pltpu.VMEM((Q,), jnp.float32),
            pltpu.VMEM((Q * 16,), jnp.int32),
            pltpu.VMEM((64,), jnp.int32),
            pltpu.VMEM((8 * Q,), jnp.float32),
            pltpu.VMEM_SHARED((NSUB * ACCW // 2,), jnp.float32),
            pltpu.VMEM_SHARED((4096,), jnp.int32),
            pltpu.VMEM_SHARED((2 * ACCW,), jnp.float32),
        ],
    )


def _mlp_body(a_ref, v_ref, w1_ref, b1_ref, w2_ref, b2_ref, out_ref):
    a = a_ref[0]
    x = lax.dot_general(a, w1_ref[...], (((0,), (0,)), ((), ())),
                        preferred_element_type=jnp.float32)
    h = jnp.maximum(x + b1_ref[0], 0.0)
    e = lax.dot_general(h, w2_ref[...], (((1,), (0,)), ((), ())),
                        preferred_element_type=jnp.float32) + b2_ref[0]
    out_ref[0] = jnp.where(v_ref[0] > 0.0, e, 0.0)


def _tc_mlp(f3, vals, w1p, b1, w2, b2):
    d = w2.shape[0]
    return pl.pallas_call(
        _mlp_body,
        grid=(F,),
        in_specs=[
            pl.BlockSpec((1, 8, Q), lambda i: (i, 0, 0)),
            pl.BlockSpec((1, Q, 1), lambda i: (i, 0, 0)),
            pl.BlockSpec((8, d), lambda i: (0, 0)),
            pl.BlockSpec((1, d), lambda i: (0, 0)),
            pl.BlockSpec((d, d), lambda i: (0, 0)),
            pl.BlockSpec((1, d), lambda i: (0, 0)),
        ],
        out_specs=pl.BlockSpec((1, Q, d), lambda i: (i, 0, 0)),
        out_shape=jax.ShapeDtypeStruct((F, Q, d), jnp.float32),
    )(f3, vals, w1p, b1, w2, b2)


def kernel(points, mask, W1, b1, W2, b2):
    pts_t = points.reshape(-1)
    mskf = mask.astype(jnp.float32).reshape(-1)
    feat8 = _sc_encode()(pts_t, mskf)
    f3 = feat8.reshape(F, 8, Q)
    d = W2.shape[0]
    w1p = jnp.concatenate([W1, jnp.zeros((3, d), jnp.float32)], axis=0)
    scores = f3[:, 4, :]
    queries = _tc_mlp(f3, scores.reshape(F, Q, 1), w1p,
                      b1.reshape(1, d), W2, b2.reshape(1, d))
    refs = jnp.transpose(f3[:, 0:3, :], (0, 2, 1))
    return queries, refs, scores


# speedup vs baseline: 3.1280x; 3.1280x over previous
import functools

import jax
import jax.numpy as jnp
from jax import lax
from jax.experimental import pallas as pl
from jax.experimental.pallas import tpu as pltpu
from jax.experimental.pallas import tpu_sc as plsc

F = 4
NPTS = 65536
NXG = 81
NYG = 81
NCELL = NXG * NYG
CPAD = 6656
NPLANE = 5
ACCW = NPLANE * CPAD
Q = 512
NSUB = 16
NCORE = 2
GROUP = 8
PPT = NPTS // GROUP
VPT = PPT // 16
NVC = CPAD // 16
X_MIN = -4.0
SX = 0.1
GMAX = NXG - 1


def _sc_body(pts_hbm, msk_hbm, out_hbm, pts_v, msk_v, acc_v, red_v, red2_v,
             hist_v, nge_v, bigkey_v, eqidx_v, sel_v, val_v, wb_v, rankf_v,
             feat8_v, sh_part, sh_aux, sh_acc):
    c = lax.axis_index("c")
    s = lax.axis_index("s")
    fc = s // GROUP
    f = c * 2 + fc
    g = s % GROUP
    p0 = g * PPT
    lanes = lax.iota(jnp.int32, 16)
    zf16 = jnp.zeros((16,), jnp.float32)
    one16 = jnp.ones((16,), jnp.float32)

    for coord in range(4):
        pltpu.sync_copy(pts_hbm.at[pl.ds(f * 4 * NPTS + coord * NPTS + p0, PPT)],
                        pts_v.at[pl.ds(coord * PPT, PPT)])
    pltpu.sync_copy(msk_hbm.at[pl.ds(f * NPTS + p0, PPT)], msk_v)

    def zero_body(i, _):
        for u in range(8):
            acc_v[pl.ds(i * 128 + u * 16, 16)] = zf16
        return 0
    lax.fori_loop(0, ACCW // 128, zero_body, 0)

    def bin_body(v, _):
      for u in range(2):
        base = v * 32 + u * 16
        px = pts_v[pl.ds(base, 16)]
        py = pts_v[pl.ds(PPT + base, 16)]
        pz = pts_v[pl.ds(2 * PPT + base, 16)]
        pi = pts_v[pl.ds(3 * PPT + base, 16)]
        m = msk_v[pl.ds(base, 16)]
        valid = ((m != 0.0)
                 & (px >= -4.0) & (px <= 4.0)
                 & (py >= -4.0) & (py <= 4.0)
                 & (pz >= -4.0) & (pz <= 4.0))
        gx = jnp.clip(((px - X_MIN) / SX).astype(jnp.int32), 0, GMAX)
        gy = jnp.clip(((py - X_MIN) / SX).astype(jnp.int32), 0, GMAX)
        cell = gx * NYG + gy
        plsc.addupdate_scatter(acc_v, [cell], one16, mask=valid)
        plsc.addupdate_scatter(acc_v, [cell + CPAD], px, mask=valid)
        plsc.addupdate_scatter(acc_v, [cell + 2 * CPAD], py, mask=valid)
        plsc.addupdate_scatter(acc_v, [cell + 3 * CPAD], pz, mask=valid)
        plsc.addupdate_scatter(acc_v, [cell + 4 * CPAD], pi, mask=valid)
      return 0
    lax.fori_loop(0, VPT // 2, bin_body, 0)

    half = ACCW // 2
    sliw = half // GROUP
    off = g * sliw
    gbase = fc * GROUP * half
    for r in range(2):
        pltpu.sync_copy(acc_v.at[pl.ds(r * half, half)],
                        sh_part.at[pl.ds(s * half, half)])
        plsc.subcore_barrier()
        pltpu.sync_copy(sh_part.at[pl.ds(gbase + off, sliw)], red2_v)
        for t in range(1, GROUP):
            pltpu.sync_copy(sh_part.at[pl.ds(gbase + t * half + off, sliw)],
                            red_v)

            def add_body(i, _):
                for u in range(5):
                    o = i * 80 + u * 16
                    red2_v[pl.ds(o, 16)] = (red2_v[pl.ds(o, 16)]
                                            + red_v[pl.ds(o, 16)])
                return 0
            lax.fori_loop(0, sliw // 80, add_body, 0)
        pltpu.sync_copy(red2_v,
                        sh_acc.at[pl.ds(fc * ACCW + r * half + off, sliw)])
        plsc.subcore_barrier()

    @pl.when(g == 0)
    def _topk():
        pltpu.sync_copy(sh_acc.at[pl.ds(fc * ACCW, ACCW)], acc_v)

        for i in range(9):
            hist_v[pl.ds(i * 16, 16)] = jnp.zeros((16,), jnp.int32)
        for i in range(32):
            bigkey_v[pl.ds(i * 16, 16)] = jnp.full((16,), -1, jnp.int32)
        for r in range(5, 8):
            def z8_body(i, _, r=r):
                feat8_v[pl.ds(r * Q + i * 16, 16)] = zf16
                return 0
            lax.fori_loop(0, Q // 16, z8_body, 0)

        def hist_body(v, _):
            for u in range(2):
                cnt = acc_v[pl.ds(v * 32 + u * 16, 16)]
                ci = cnt.astype(jnp.int32)
                bin_ = jnp.minimum(ci, 128)
                cellidx = v * 32 + u * 16 + lanes
                plsc.addupdate_scatter(hist_v, [bin_],
                                       jnp.ones((16,), jnp.int32),
                                       mask=cellidx < NCELL)
            return 0
        lax.fori_loop(0, NVC // 2, hist_body, 0)

        carry = jnp.zeros((16,), jnp.int32)
        for vi in range(8, -1, -1):
            h = hist_v[pl.ds(vi * 16, 16)]
            cs = lax.rev(plsc.cumsum(lax.rev(h, (0,))), (0,))
            nge_v[pl.ds(vi * 16, 16)] = cs + carry
            carry = carry + jnp.full((16,), jnp.sum(h))

        T = jnp.int32(-1)
        for vi in range(9):
            tvec = lanes + vi * 16
            ngev = nge_v[pl.ds(vi * 16, 16)]
            T = jnp.maximum(T, jnp.max(jnp.where(ngev >= Q, tvec, -1)))
        M = jnp.int32(0)
        for vi in range(9):
            tvec = lanes + vi * 16
            ngev = nge_v[pl.ds(vi * 16, 16)]
            M = jnp.maximum(M, jnp.max(jnp.where(tvec == T + 1, ngev, 0)))

        def comp_body(v, bases):
            bigbase, eqbase = bases
            for u in range(2):
                ci = acc_v[pl.ds(v * 32 + u * 16, 16)].astype(jnp.int32)
                cellidx = v * 32 + u * 16 + lanes
                iscell = cellidx < NCELL
                big = (ci > T) & iscell
                eq = (ci == T) & iscell
                key = ci * 8192 + (8191 - cellidx)
                bpos = bigbase + plsc.cumsum(big.astype(jnp.int32)) - 1
                plsc.store_scatter(bigkey_v, [bpos], key, mask=big)
                epos = eqbase + plsc.cumsum(eq.astype(jnp.int32)) - 1
                plsc.store_scatter(eqidx_v, [epos], cellidx, mask=eq)
                nb = plsc.all_reduce_population_count(big).astype(jnp.int32)
                ne = plsc.all_reduce_population_count(eq).astype(jnp.int32)
                bigbase = bigbase + nb
                eqbase = eqbase + ne
            return (bigbase, eqbase)
        lax.fori_loop(0, NVC // 2, comp_body,
                      (jnp.zeros((16,), jnp.int32), jnp.zeros((16,), jnp.int32)))

        pltpu.sync_copy(bigkey_v, sh_aux.at[pl.ds(fc * 2048, Q)])
        rankf_v[pl.ds(0, 16)] = jnp.full((16,), M)
        rankf_v[pl.ds(16, 16)] = jnp.full((16,), T)
        pltpu.sync_copy(rankf_v.at[pl.ds(0, 32)],
                        sh_aux.at[pl.ds(fc * 2048 + 512, 32)])

    plsc.subcore_barrier()
    pltpu.sync_copy(sh_aux.at[pl.ds(fc * 2048, Q)], bigkey_v)

    def wb_body(j, _):
        wv = bigkey_v[pl.ds(j * 16, 16)]
        for l in range(16):
            wl = jnp.max(jnp.where(lanes == l, wv, jnp.int32(-2**31 + 1)))
            wb_v[pl.ds((j * 16 + l) * 16, 16)] = jnp.full((16,), wl)
        return 0
    lax.fori_loop(0, 32, wb_body, 0)

    for u in range(4):
        kv = bigkey_v[pl.ds((g * 4 + u) * 16, 16)]

        def cnt_body(j8, r, kv=kv):
            for u8 in range(8):
                r = r + (wb_v[pl.ds(j8 * 128 + u8 * 16, 16)] > kv
                         ).astype(jnp.int32)
            return r
        rankv = lax.fori_loop(0, Q // 8, cnt_body, jnp.zeros((16,), jnp.int32))
        rankf_v[pl.ds(u * 16, 16)] = rankv
    pltpu.sync_copy(rankf_v, sh_aux.at[pl.ds(fc * 2048 + 1024 + g * 64, 64)])
    plsc.subcore_barrier()

    @pl.when(g == 0)
    def _emit():
        pltpu.sync_copy(sh_aux.at[pl.ds(fc * 2048 + 1024, Q)],
                        wb_v.at[pl.ds(0, Q)])
        pltpu.sync_copy(sh_aux.at[pl.ds(fc * 2048 + 512, 32)],
                        rankf_v.at[pl.ds(0, 32)])
        M = jnp.max(rankf_v[pl.ds(0, 16)])
        T = jnp.max(rankf_v[pl.ds(16, 16)])

        def scat_body(i, _):
            kv = bigkey_v[pl.ds(i * 16, 16)]
            rankv = wb_v[pl.ds(i * 16, 16)]
            mb = (i * 16 + lanes) < M
            idx = 8191 - (kv & 8191)
            valf = (kv >> 13).astype(jnp.float32)
            plsc.store_scatter(sel_v, [rankv], idx, mask=mb)
            plsc.store_scatter(val_v, [rankv], valf, mask=mb)
            return 0
        lax.fori_loop(0, 32, scat_body, 0)

        def eq_body(v, _):
            t = v * 16 + lanes
            me = t < (Q - M)
            eidx = eqidx_v[pl.ds(v * 16, 16)]
            plsc.store_scatter(sel_v, [M + t], eidx, mask=me)
            plsc.store_scatter(val_v, [M + t],
                              jnp.full((16,), T.astype(jnp.float32)), mask=me)
            return 0
        lax.fori_loop(0, Q // 16, eq_body, 0)

        def gat_body(v, _):
            base = v * 16
            idxv = sel_v[pl.ds(base, 16)]
            valv = val_v[pl.ds(base, 16)]
            denom = jnp.maximum(valv, 1.0)
            for r in range(4):
                sr = plsc.load_gather(acc_v, [idxv + (r + 1) * CPAD])
                feat8_v[pl.ds(r * Q + base, 16)] = sr / denom
            feat8_v[pl.ds(4 * Q + base, 16)] = valv
            return 0
        lax.fori_loop(0, Q // 16, gat_body, 0)

        pltpu.sync_copy(feat8_v, out_hbm.at[pl.ds(f * 8 * Q, 8 * Q)])


@functools.lru_cache(maxsize=1)
def _sc_encode():
    mesh = plsc.VectorSubcoreMesh(core_axis_name="c", subcore_axis_name="s",
                                  num_cores=NCORE, num_subcores=NSUB)
    return pl.kernel(
        _sc_body,
        out_type=jax.ShapeDtypeStruct((F * 8 * Q,), jnp.float32),
        mesh=mesh,
        compiler_params=pltpu.CompilerParams(needs_layout_passes=False),
        scratch_types=[
            pltpu.VMEM((4 * PPT,), jnp.float32),
            pltpu.VMEM((PPT,), jnp.float32),
            pltpu.VMEM((ACCW,), jnp.float32),
            pltpu.VMEM((ACCW // 2 // GROUP,), jnp.float32),
            pltpu.VMEM((ACCW // 2 // GROUP,), jnp.float32),
            pltpu.VMEM((144,), jnp.int32),
            pltpu.VMEM((144,), jnp.int32),
            pltpu.VMEM((Q,), jnp.int32),
            pltpu.VMEM((CPAD,), jnp.int32),
            pltpu.VMEM((Q,), jnp.int32),
            pltpu.VMEM((Q,), jnp.float32),
            pltpu.VMEM((Q * 16,), jnp.int32),
            pltpu.VMEM((64,), jnp.int32),
            pltpu.VMEM((8 * Q,), jnp.float32),
            pltpu.VMEM_SHARED((NSUB * ACCW // 2,), jnp.float32),
            pltpu.VMEM_SHARED((4096,), jnp.int32),
            pltpu.VMEM_SHARED((2 * ACCW,), jnp.float32),
        ],
    )


def _mlp_body(a_ref, v_ref, w1_ref, b1_ref, w2_ref, b2_ref, out_ref):
    a = a_ref[0]
    x = lax.dot_general(a, w1_ref[...], (((0,), (0,)), ((), ())),
                        preferred_element_type=jnp.float32)
    h = jnp.maximum(x + b1_ref[0], 0.0)
    e = lax.dot_general(h, w2_ref[...], (((1,), (0,)), ((), ())),
                        preferred_element_type=jnp.float32) + b2_ref[0]
    out_ref[0] = jnp.where(v_ref[0] > 0.0, e, 0.0)


def _tc_mlp(f3, vals, w1p, b1, w2, b2):
    d = w2.shape[0]
    return pl.pallas_call(
        _mlp_body,
        grid=(F,),
        in_specs=[
            pl.BlockSpec((1, 8, Q), lambda i: (i, 0, 0)),
            pl.BlockSpec((1, Q, 1), lambda i: (i, 0, 0)),
            pl.BlockSpec((8, d), lambda i: (0, 0)),
            pl.BlockSpec((1, d), lambda i: (0, 0)),
            pl.BlockSpec((d, d), lambda i: (0, 0)),
            pl.BlockSpec((1, d), lambda i: (0, 0)),
        ],
        out_specs=pl.BlockSpec((1, Q, d), lambda i: (i, 0, 0)),
        out_shape=jax.ShapeDtypeStruct((F, Q, d), jnp.float32),
    )(f3, vals, w1p, b1, w2, b2)


def kernel(points, mask, W1, b1, W2, b2):
    pts_t = jnp.transpose(points, (0, 2, 1)).reshape(-1)
    mskf = mask.astype(jnp.float32).reshape(-1)
    feat8 = _sc_encode()(pts_t, mskf)
    f3 = feat8.reshape(F, 8, Q)
    d = W2.shape[0]
    w1p = jnp.concatenate([W1, jnp.zeros((3, d), jnp.float32)], axis=0)
    scores = f3[:, 4, :]
    queries = _tc_mlp(f3, scores.reshape(F, Q, 1), w1p,
                      b1.reshape(1, d), W2, b2.reshape(1, d))
    refs = jnp.transpose(f3[:, 0:3, :], (0, 2, 1))
    return queries, refs, scores


# speedup vs baseline: 3.1963x; 1.0218x over previous
import functools

import jax
import jax.numpy as jnp
from jax import lax
from jax.experimental import pallas as pl
from jax.experimental.pallas import tpu as pltpu
from jax.experimental.pallas import tpu_sc as plsc

F = 4
NPTS = 65536
NXG = 81
NYG = 81
NCELL = NXG * NYG
CPAD = 6656
NPLANE = 5
ACCW = NPLANE * CPAD
Q = 512
NSUB = 16
NCORE = 2
GROUP = 8
PPT = NPTS // GROUP
VPT = PPT // 16
NVC = CPAD // 16
X_MIN = -4.0
SX = 0.1
GMAX = NXG - 1


def _sc_body(pts_hbm, out_hbm, pts_v, acc_v, red_v, red2_v,
             hist_v, nge_v, bigkey_v, eqidx_v, sel_v, val_v, wb_v, rankf_v,
             feat8_v, sh_part, sh_aux, sh_acc):
    c = lax.axis_index("c")
    s = lax.axis_index("s")
    fc = s // GROUP
    f = c * 2 + fc
    g = s % GROUP
    p0 = g * PPT
    lanes = lax.iota(jnp.int32, 16)
    zf16 = jnp.zeros((16,), jnp.float32)
    one16 = jnp.ones((16,), jnp.float32)

    for coord in range(4):
        pltpu.sync_copy(pts_hbm.at[pl.ds(f * 4 * NPTS + coord * NPTS + p0, PPT)],
                        pts_v.at[pl.ds(coord * PPT, PPT)])

    def zero_body(i, _):
        for u in range(8):
            acc_v[pl.ds(i * 128 + u * 16, 16)] = zf16
        return 0
    lax.fori_loop(0, ACCW // 128, zero_body, 0)

    def bin_body(v, _):
      for u in range(2):
        base = v * 32 + u * 16
        px = pts_v[pl.ds(base, 16)]
        py = pts_v[pl.ds(PPT + base, 16)]
        pz = pts_v[pl.ds(2 * PPT + base, 16)]
        pi = pts_v[pl.ds(3 * PPT + base, 16)]
        valid = ((px >= -4.0) & (px <= 4.0)
                 & (py >= -4.0) & (py <= 4.0)
                 & (pz >= -4.0) & (pz <= 4.0))
        gx = jnp.clip(((px - X_MIN) / SX).astype(jnp.int32), 0, GMAX)
        gy = jnp.clip(((py - X_MIN) / SX).astype(jnp.int32), 0, GMAX)
        cell = gx * NYG + gy
        plsc.addupdate_scatter(acc_v, [cell], one16, mask=valid)
        plsc.addupdate_scatter(acc_v, [cell + CPAD], px, mask=valid)
        plsc.addupdate_scatter(acc_v, [cell + 2 * CPAD], py, mask=valid)
        plsc.addupdate_scatter(acc_v, [cell + 3 * CPAD], pz, mask=valid)
        plsc.addupdate_scatter(acc_v, [cell + 4 * CPAD], pi, mask=valid)
      return 0
    lax.fori_loop(0, VPT // 2, bin_body, 0)

    half = ACCW // 2
    sliw = half // GROUP
    off = g * sliw
    gbase = fc * GROUP * half
    for r in range(2):
        pltpu.sync_copy(acc_v.at[pl.ds(r * half, half)],
                        sh_part.at[pl.ds(s * half, half)])
        plsc.subcore_barrier()
        pltpu.sync_copy(sh_part.at[pl.ds(gbase + off, sliw)], red2_v)
        for t in range(1, GROUP):
            pltpu.sync_copy(sh_part.at[pl.ds(gbase + t * half + off, sliw)],
                            red_v)

            def add_body(i, _):
                for u in range(5):
                    o = i * 80 + u * 16
                    red2_v[pl.ds(o, 16)] = (red2_v[pl.ds(o, 16)]
                                            + red_v[pl.ds(o, 16)])
                return 0
            lax.fori_loop(0, sliw // 80, add_body, 0)
        pltpu.sync_copy(red2_v,
                        sh_acc.at[pl.ds(fc * ACCW + r * half + off, sliw)])
        plsc.subcore_barrier()

    @pl.when(g == 0)
    def _topk():
        pltpu.sync_copy(sh_acc.at[pl.ds(fc * ACCW, ACCW)], acc_v)

        for i in range(9):
            hist_v[pl.ds(i * 16, 16)] = jnp.zeros((16,), jnp.int32)
        for i in range(32):
            bigkey_v[pl.ds(i * 16, 16)] = jnp.full((16,), -1, jnp.int32)
        for r in range(5, 8):
            def z8_body(i, _, r=r):
                feat8_v[pl.ds(r * Q + i * 16, 16)] = zf16
                return 0
            lax.fori_loop(0, Q // 16, z8_body, 0)

        def hist_body(v, _):
            for u in range(2):
                cnt = acc_v[pl.ds(v * 32 + u * 16, 16)]
                ci = cnt.astype(jnp.int32)
                bin_ = jnp.minimum(ci, 128)
                cellidx = v * 32 + u * 16 + lanes
                plsc.addupdate_scatter(hist_v, [bin_],
                                       jnp.ones((16,), jnp.int32),
                                       mask=cellidx < NCELL)
            return 0
        lax.fori_loop(0, NVC // 2, hist_body, 0)

        carry = jnp.zeros((16,), jnp.int32)
        for vi in range(8, -1, -1):
            h = hist_v[pl.ds(vi * 16, 16)]
            cs = lax.rev(plsc.cumsum(lax.rev(h, (0,))), (0,))
            nge_v[pl.ds(vi * 16, 16)] = cs + carry
            carry = carry + jnp.full((16,), jnp.sum(h))

        T = jnp.int32(-1)
        for vi in range(9):
            tvec = lanes + vi * 16
            ngev = nge_v[pl.ds(vi * 16, 16)]
            T = jnp.maximum(T, jnp.max(jnp.where(ngev >= Q, tvec, -1)))
        M = jnp.int32(0)
        for vi in range(9):
            tvec = lanes + vi * 16
            ngev = nge_v[pl.ds(vi * 16, 16)]
            M = jnp.maximum(M, jnp.max(jnp.where(tvec == T + 1, ngev, 0)))

        def comp_body(v, bases):
            bigbase, eqbase = bases
            for u in range(2):
                ci = acc_v[pl.ds(v * 32 + u * 16, 16)].astype(jnp.int32)
                cellidx = v * 32 + u * 16 + lanes
                iscell = cellidx < NCELL
                big = (ci > T) & iscell
                eq = (ci == T) & iscell
                key = ci * 8192 + (8191 - cellidx)
                bpos = bigbase + plsc.cumsum(big.astype(jnp.int32)) - 1
                plsc.store_scatter(bigkey_v, [bpos], key, mask=big)
                epos = eqbase + plsc.cumsum(eq.astype(jnp.int32)) - 1
                plsc.store_scatter(eqidx_v, [epos], cellidx, mask=eq)
                nb = plsc.all_reduce_population_count(big).astype(jnp.int32)
                ne = plsc.all_reduce_population_count(eq).astype(jnp.int32)
                bigbase = bigbase + nb
                eqbase = eqbase + ne
            return (bigbase, eqbase)
        lax.fori_loop(0, NVC // 2, comp_body,
                      (jnp.zeros((16,), jnp.int32), jnp.zeros((16,), jnp.int32)))

        pltpu.sync_copy(bigkey_v, sh_aux.at[pl.ds(fc * 2048, Q)])
        rankf_v[pl.ds(0, 16)] = jnp.full((16,), M)
        rankf_v[pl.ds(16, 16)] = jnp.full((16,), T)
        pltpu.sync_copy(rankf_v.at[pl.ds(0, 32)],
                        sh_aux.at[pl.ds(fc * 2048 + 512, 32)])

    plsc.subcore_barrier()
    pltpu.sync_copy(sh_aux.at[pl.ds(fc * 2048, Q)], bigkey_v)

    def wb_body(j, _):
        wv = bigkey_v[pl.ds(j * 16, 16)]
        for l in range(16):
            wl = jnp.max(jnp.where(lanes == l, wv, jnp.int32(-2**31 + 1)))
            wb_v[pl.ds((j * 16 + l) * 16, 16)] = jnp.full((16,), wl)
        return 0
    lax.fori_loop(0, 32, wb_body, 0)

    for u in range(4):
        kv = bigkey_v[pl.ds((g * 4 + u) * 16, 16)]

        def cnt_body(j8, r, kv=kv):
            for u8 in range(8):
                r = r + (wb_v[pl.ds(j8 * 128 + u8 * 16, 16)] > kv
                         ).astype(jnp.int32)
            return r
        rankv = lax.fori_loop(0, Q // 8, cnt_body, jnp.zeros((16,), jnp.int32))
        rankf_v[pl.ds(u * 16, 16)] = rankv
    pltpu.sync_copy(rankf_v, sh_aux.at[pl.ds(fc * 2048 + 1024 + g * 64, 64)])
    plsc.subcore_barrier()

    @pl.when(g == 0)
    def _emit():
        pltpu.sync_copy(sh_aux.at[pl.ds(fc * 2048 + 1024, Q)],
                        wb_v.at[pl.ds(0, Q)])
        pltpu.sync_copy(sh_aux.at[pl.ds(fc * 2048 + 512, 32)],
                        rankf_v.at[pl.ds(0, 32)])
        M = jnp.max(rankf_v[pl.ds(0, 16)])
        T = jnp.max(rankf_v[pl.ds(16, 16)])

        def scat_body(i, _):
            kv = bigkey_v[pl.ds(i * 16, 16)]
            rankv = wb_v[pl.ds(i * 16, 16)]
            mb = (i * 16 + lanes) < M
            idx = 8191 - (kv & 8191)
            valf = (kv >> 13).astype(jnp.float32)
            plsc.store_scatter(sel_v, [rankv], idx, mask=mb)
            plsc.store_scatter(val_v, [rankv], valf, mask=mb)
            return 0
        lax.fori_loop(0, 32, scat_body, 0)

        def eq_body(v, _):
            t = v * 16 + lanes
            me = t < (Q - M)
            eidx = eqidx_v[pl.ds(v * 16, 16)]
            plsc.store_scatter(sel_v, [M + t], eidx, mask=me)
            plsc.store_scatter(val_v, [M + t],
                              jnp.full((16,), T.astype(jnp.float32)), mask=me)
            return 0
        lax.fori_loop(0, Q // 16, eq_body, 0)

        def gat_body(v, _):
            base = v * 16
            idxv = sel_v[pl.ds(base, 16)]
            valv = val_v[pl.ds(base, 16)]
            denom = jnp.maximum(valv, 1.0)
            for r in range(4):
                sr = plsc.load_gather(acc_v, [idxv + (r + 1) * CPAD])
                feat8_v[pl.ds(r * Q + base, 16)] = sr / denom
            feat8_v[pl.ds(4 * Q + base, 16)] = valv
            return 0
        lax.fori_loop(0, Q // 16, gat_body, 0)

        pltpu.sync_copy(feat8_v, out_hbm.at[pl.ds(f * 8 * Q, 8 * Q)])


@functools.lru_cache(maxsize=1)
def _sc_encode():
    mesh = plsc.VectorSubcoreMesh(core_axis_name="c", subcore_axis_name="s",
                                  num_cores=NCORE, num_subcores=NSUB)
    return pl.kernel(
        _sc_body,
        out_type=jax.ShapeDtypeStruct((F * 8 * Q,), jnp.float32),
        mesh=mesh,
        compiler_params=pltpu.CompilerParams(needs_layout_passes=False),
        scratch_types=[
            pltpu.VMEM((4 * PPT,), jnp.float32),
            pltpu.VMEM((ACCW,), jnp.float32),
            pltpu.VMEM((ACCW // 2 // GROUP,), jnp.float32),
            pltpu.VMEM((ACCW // 2 // GROUP,), jnp.float32),
            pltpu.VMEM((144,), jnp.int32),
            pltpu.VMEM((144,), jnp.int32),
            pltpu.VMEM((Q,), jnp.int32),
            pltpu.VMEM((CPAD,), jnp.int32),
            pltpu.VMEM((Q,), jnp.int32),
            pltpu.VMEM((Q,), jnp.float32),
            pltpu.VMEM((Q * 16,), jnp.int32),
            pltpu.VMEM((64,), jnp.int32),
            pltpu.VMEM((8 * Q,), jnp.float32),
            pltpu.VMEM_SHARED((NSUB * ACCW // 2,), jnp.float32),
            pltpu.VMEM_SHARED((4096,), jnp.int32),
            pltpu.VMEM_SHARED((2 * ACCW,), jnp.float32),
        ],
    )


def _mlp_body(a_ref, v_ref, w1_ref, b1_ref, w2_ref, b2_ref, out_ref):
    a = a_ref[0]
    x = lax.dot_general(a, w1_ref[...], (((0,), (0,)), ((), ())),
                        preferred_element_type=jnp.float32)
    h = jnp.maximum(x + b1_ref[0], 0.0)
    e = lax.dot_general(h, w2_ref[...], (((1,), (0,)), ((), ())),
                        preferred_element_type=jnp.float32) + b2_ref[0]
    out_ref[0] = jnp.where(v_ref[0] > 0.0, e, 0.0)


def _tc_mlp(f3, vals, w1p, b1, w2, b2):
    d = w2.shape[0]
    return pl.pallas_call(
        _mlp_body,
        grid=(F,),
        in_specs=[
            pl.BlockSpec((1, 8, Q), lambda i: (i, 0, 0)),
            pl.BlockSpec((1, Q, 1), lambda i: (i, 0, 0)),
            pl.BlockSpec((8, d), lambda i: (0, 0)),
            pl.BlockSpec((1, d), lambda i: (0, 0)),
            pl.BlockSpec((d, d), lambda i: (0, 0)),
            pl.BlockSpec((1, d), lambda i: (0, 0)),
        ],
        out_specs=pl.BlockSpec((1, Q, d), lambda i: (i, 0, 0)),
        out_shape=jax.ShapeDtypeStruct((F, Q, d), jnp.float32),
    )(f3, vals, w1p, b1, w2, b2)


def kernel(points, mask, W1, b1, W2, b2):
    pts_m = jnp.where(mask[:, :, None], points, 1e9)
    pts_t = jnp.transpose(pts_m, (0, 2, 1)).reshape(-1)
    feat8 = _sc_encode()(pts_t)
    f3 = feat8.reshape(F, 8, Q)
    d = W2.shape[0]
    w1p = jnp.concatenate([W1, jnp.zeros((3, d), jnp.float32)], axis=0)
    scores = f3[:, 4, :]
    queries = _tc_mlp(f3, scores.reshape(F, Q, 1), w1p,
                      b1.reshape(1, d), W2, b2.reshape(1, d))
    refs = jnp.transpose(f3[:, 0:3, :], (0, 2, 1))
    return queries, refs, scores


# speedup vs baseline: 3.3350x; 1.0434x over previous
import functools

import jax
import jax.numpy as jnp
from jax import lax
from jax.experimental import pallas as pl
from jax.experimental.pallas import tpu as pltpu
from jax.experimental.pallas import tpu_sc as plsc

F = 4
NPTS = 65536
NXG = 81
NYG = 81
NCELL = NXG * NYG
CPAD = 6656
NPLANE = 5
ACCW = NPLANE * CPAD
Q = 512
NSUB = 16
NCORE = 2
GROUP = 8
PPT = NPTS // GROUP
VPT = PPT // 16
NVC = CPAD // 16
X_MIN = -4.0
SX = 0.1
GMAX = NXG - 1


def _sc_body(pts_hbm, out_hbm, pts_v, acc_v, red_v, red2_v,
             hist_v, nge_v, bigkey_v, eqidx_v, sel_v, val_v, wb_v, rankf_v,
             feat8_v, dma_sem, sh_part, sh_aux, sh_acc):
    c = lax.axis_index("c")
    s = lax.axis_index("s")
    fc = s // GROUP
    f = c * 2 + fc
    g = s % GROUP
    p0 = g * PPT
    lanes = lax.iota(jnp.int32, 16)
    zf16 = jnp.zeros((16,), jnp.float32)
    one16 = jnp.ones((16,), jnp.float32)

    descs = [
        pltpu.async_copy(
            pts_hbm.at[pl.ds(f * 4 * NPTS + coord * NPTS + p0, PPT)],
            pts_v.at[pl.ds(coord * PPT, PPT)], dma_sem)
        for coord in range(4)
    ]

    def zero_body(i, _):
        for u in range(8):
            acc_v[pl.ds(i * 128 + u * 16, 16)] = zf16
        return 0
    lax.fori_loop(0, ACCW // 128, zero_body, 0)
    for d in descs:
        d.wait()

    def bin_body(v, _):
      for u in range(2):
        base = v * 32 + u * 16
        px = pts_v[pl.ds(base, 16)]
        py = pts_v[pl.ds(PPT + base, 16)]
        pz = pts_v[pl.ds(2 * PPT + base, 16)]
        pi = pts_v[pl.ds(3 * PPT + base, 16)]
        valid = ((px >= -4.0) & (px <= 4.0)
                 & (py >= -4.0) & (py <= 4.0)
                 & (pz >= -4.0) & (pz <= 4.0))
        gx = jnp.clip(((px - X_MIN) / SX).astype(jnp.int32), 0, GMAX)
        gy = jnp.clip(((py - X_MIN) / SX).astype(jnp.int32), 0, GMAX)
        cell = gx * NYG + gy
        plsc.addupdate_scatter(acc_v, [cell], one16, mask=valid)
        plsc.addupdate_scatter(acc_v, [cell + CPAD], px, mask=valid)
        plsc.addupdate_scatter(acc_v, [cell + 2 * CPAD], py, mask=valid)
        plsc.addupdate_scatter(acc_v, [cell + 3 * CPAD], pz, mask=valid)
        plsc.addupdate_scatter(acc_v, [cell + 4 * CPAD], pi, mask=valid)
      return 0
    lax.fori_loop(0, VPT // 2, bin_body, 0)

    half = ACCW // 2
    sliw = half // GROUP
    off = g * sliw
    gbase = fc * GROUP * half
    for r in range(2):
        pltpu.sync_copy(acc_v.at[pl.ds(r * half, half)],
                        sh_part.at[pl.ds(s * half, half)])
        plsc.subcore_barrier()
        pltpu.sync_copy(sh_part.at[pl.ds(gbase + off, sliw)], red2_v)
        for t in range(1, GROUP):
            pltpu.sync_copy(sh_part.at[pl.ds(gbase + t * half + off, sliw)],
                            red_v)

            def add_body(i, _):
                for u in range(5):
                    o = i * 80 + u * 16
                    red2_v[pl.ds(o, 16)] = (red2_v[pl.ds(o, 16)]
                                            + red_v[pl.ds(o, 16)])
                return 0
            lax.fori_loop(0, sliw // 80, add_body, 0)
        pltpu.sync_copy(red2_v,
                        sh_acc.at[pl.ds(fc * ACCW + r * half + off, sliw)])
        plsc.subcore_barrier()

    @pl.when(g == 0)
    def _topk():
        pltpu.sync_copy(sh_acc.at[pl.ds(fc * ACCW, ACCW)], acc_v)

        for i in range(9):
            hist_v[pl.ds(i * 16, 16)] = jnp.zeros((16,), jnp.int32)
        for i in range(32):
            bigkey_v[pl.ds(i * 16, 16)] = jnp.full((16,), -1, jnp.int32)
        for r in range(5, 8):
            def z8_body(i, _, r=r):
                feat8_v[pl.ds(r * Q + i * 16, 16)] = zf16
                return 0
            lax.fori_loop(0, Q // 16, z8_body, 0)

        def hist_body(v, _):
            for u in range(2):
                cnt = acc_v[pl.ds(v * 32 + u * 16, 16)]
                ci = cnt.astype(jnp.int32)
                bin_ = jnp.minimum(ci, 128)
                cellidx = v * 32 + u * 16 + lanes
                plsc.addupdate_scatter(hist_v, [bin_],
                                       jnp.ones((16,), jnp.int32),
                                       mask=cellidx < NCELL)
            return 0
        lax.fori_loop(0, NVC // 2, hist_body, 0)

        carry = jnp.zeros((16,), jnp.int32)
        for vi in range(8, -1, -1):
            h = hist_v[pl.ds(vi * 16, 16)]
            cs = lax.rev(plsc.cumsum(lax.rev(h, (0,))), (0,))
            nge_v[pl.ds(vi * 16, 16)] = cs + carry
            carry = carry + jnp.full((16,), jnp.sum(h))

        T = jnp.int32(-1)
        for vi in range(9):
            tvec = lanes + vi * 16
            ngev = nge_v[pl.ds(vi * 16, 16)]
            T = jnp.maximum(T, jnp.max(jnp.where(ngev >= Q, tvec, -1)))
        M = jnp.int32(0)
        for vi in range(9):
            tvec = lanes + vi * 16
            ngev = nge_v[pl.ds(vi * 16, 16)]
            M = jnp.maximum(M, jnp.max(jnp.where(tvec == T + 1, ngev, 0)))

        def comp_body(v, bases):
            bigbase, eqbase = bases
            for u in range(2):
                ci = acc_v[pl.ds(v * 32 + u * 16, 16)].astype(jnp.int32)
                cellidx = v * 32 + u * 16 + lanes
                iscell = cellidx < NCELL
                big = (ci > T) & iscell
                eq = (ci == T) & iscell
                key = ci * 8192 + (8191 - cellidx)
                bpos = bigbase + plsc.cumsum(big.astype(jnp.int32)) - 1
                plsc.store_scatter(bigkey_v, [bpos], key, mask=big)
                epos = eqbase + plsc.cumsum(eq.astype(jnp.int32)) - 1
                plsc.store_scatter(eqidx_v, [epos], cellidx, mask=eq)
                nb = plsc.all_reduce_population_count(big).astype(jnp.int32)
                ne = plsc.all_reduce_population_count(eq).astype(jnp.int32)
                bigbase = bigbase + nb
                eqbase = eqbase + ne
            return (bigbase, eqbase)
        lax.fori_loop(0, NVC // 2, comp_body,
                      (jnp.zeros((16,), jnp.int32), jnp.zeros((16,), jnp.int32)))

        pltpu.sync_copy(bigkey_v, sh_aux.at[pl.ds(fc * 2048, Q)])
        rankf_v[pl.ds(0, 16)] = jnp.full((16,), M)
        rankf_v[pl.ds(16, 16)] = jnp.full((16,), T)
        pltpu.sync_copy(rankf_v.at[pl.ds(0, 32)],
                        sh_aux.at[pl.ds(fc * 2048 + 512, 32)])

    plsc.subcore_barrier()
    pltpu.sync_copy(sh_aux.at[pl.ds(fc * 2048, Q)], bigkey_v)

    def wb_body(j, _):
        wv = bigkey_v[pl.ds(j * 16, 16)]
        for l in range(16):
            wl = jnp.max(jnp.where(lanes == l, wv, jnp.int32(-2**31 + 1)))
            wb_v[pl.ds((j * 16 + l) * 16, 16)] = jnp.full((16,), wl)
        return 0
    lax.fori_loop(0, 32, wb_body, 0)

    for u in range(4):
        kv = bigkey_v[pl.ds((g * 4 + u) * 16, 16)]

        def cnt_body(j8, r, kv=kv):
            for u8 in range(8):
                r = r + (wb_v[pl.ds(j8 * 128 + u8 * 16, 16)] > kv
                         ).astype(jnp.int32)
            return r
        rankv = lax.fori_loop(0, Q // 8, cnt_body, jnp.zeros((16,), jnp.int32))
        rankf_v[pl.ds(u * 16, 16)] = rankv
    pltpu.sync_copy(rankf_v, sh_aux.at[pl.ds(fc * 2048 + 1024 + g * 64, 64)])
    plsc.subcore_barrier()

    @pl.when(g == 0)
    def _emit():
        pltpu.sync_copy(sh_aux.at[pl.ds(fc * 2048 + 1024, Q)],
                        wb_v.at[pl.ds(0, Q)])
        pltpu.sync_copy(sh_aux.at[pl.ds(fc * 2048 + 512, 32)],
                        rankf_v.at[pl.ds(0, 32)])
        M = jnp.max(rankf_v[pl.ds(0, 16)])
        T = jnp.max(rankf_v[pl.ds(16, 16)])

        def scat_body(i, _):
            kv = bigkey_v[pl.ds(i * 16, 16)]
            rankv = wb_v[pl.ds(i * 16, 16)]
            mb = (i * 16 + lanes) < M
            idx = 8191 - (kv & 8191)
            valf = (kv >> 13).astype(jnp.float32)
            plsc.store_scatter(sel_v, [rankv], idx, mask=mb)
            plsc.store_scatter(val_v, [rankv], valf, mask=mb)
            return 0
        lax.fori_loop(0, 32, scat_body, 0)

        def eq_body(v, _):
            t = v * 16 + lanes
            me = t < (Q - M)
            eidx = eqidx_v[pl.ds(v * 16, 16)]
            plsc.store_scatter(sel_v, [M + t], eidx, mask=me)
            plsc.store_scatter(val_v, [M + t],
                              jnp.full((16,), T.astype(jnp.float32)), mask=me)
            return 0
        lax.fori_loop(0, Q // 16, eq_body, 0)

        def gat_body(v, _):
            base = v * 16
            idxv = sel_v[pl.ds(base, 16)]
            valv = val_v[pl.ds(base, 16)]
            denom = jnp.maximum(valv, 1.0)
            for r in range(4):
                sr = plsc.load_gather(acc_v, [idxv + (r + 1) * CPAD])
                feat8_v[pl.ds(r * Q + base, 16)] = sr / denom
            feat8_v[pl.ds(4 * Q + base, 16)] = valv
            return 0
        lax.fori_loop(0, Q // 16, gat_body, 0)

        pltpu.sync_copy(feat8_v, out_hbm.at[pl.ds(f * 8 * Q, 8 * Q)])


@functools.lru_cache(maxsize=1)
def _sc_encode():
    mesh = plsc.VectorSubcoreMesh(core_axis_name="c", subcore_axis_name="s",
                                  num_cores=NCORE, num_subcores=NSUB)
    return pl.kernel(
        _sc_body,
        out_type=jax.ShapeDtypeStruct((F * 8 * Q,), jnp.float32),
        mesh=mesh,
        compiler_params=pltpu.CompilerParams(needs_layout_passes=False),
        scratch_types=[
            pltpu.VMEM((4 * PPT,), jnp.float32),
            pltpu.VMEM((ACCW,), jnp.float32),
            pltpu.VMEM((ACCW // 2 // GROUP,), jnp.float32),
            pltpu.VMEM((ACCW // 2 // GROUP,), jnp.float32),
            pltpu.VMEM((144,), jnp.int32),
            pltpu.VMEM((144,), jnp.int32),
            pltpu.VMEM((Q,), jnp.int32),
            pltpu.VMEM((CPAD,), jnp.int32),
            pltpu.VMEM((Q,), jnp.int32),
            pltpu.VMEM((Q,), jnp.float32),
            pltpu.VMEM((Q * 16,), jnp.int32),
            pltpu.VMEM((64,), jnp.int32),
            pltpu.VMEM((8 * Q,), jnp.float32),
            pltpu.SemaphoreType.DMA,
            pltpu.VMEM_SHARED((NSUB * ACCW // 2,), jnp.float32),
            pltpu.VMEM_SHARED((4096,), jnp.int32),
            pltpu.VMEM_SHARED((2 * ACCW,), jnp.float32),
        ],
    )


def _mlp_body(a_ref, v_ref, w1_ref, b1_ref, w2_ref, b2_ref, out_ref):
    a = a_ref[0]
    x = lax.dot_general(a, w1_ref[...], (((0,), (0,)), ((), ())),
                        preferred_element_type=jnp.float32)
    h = jnp.maximum(x + b1_ref[0], 0.0)
    e = lax.dot_general(h, w2_ref[...], (((1,), (0,)), ((), ())),
                        preferred_element_type=jnp.float32) + b2_ref[0]
    out_ref[0] = jnp.where(v_ref[0] > 0.0, e, 0.0)


def _tc_mlp(f3, vals, w1p, b1, w2, b2):
    d = w2.shape[0]
    return pl.pallas_call(
        _mlp_body,
        grid=(F,),
        in_specs=[
            pl.BlockSpec((1, 8, Q), lambda i: (i, 0, 0)),
            pl.BlockSpec((1, Q, 1), lambda i: (i, 0, 0)),
            pl.BlockSpec((8, d), lambda i: (0, 0)),
            pl.BlockSpec((1, d), lambda i: (0, 0)),
            pl.BlockSpec((d, d), lambda i: (0, 0)),
            pl.BlockSpec((1, d), lambda i: (0, 0)),
        ],
        out_specs=pl.BlockSpec((1, Q, d), lambda i: (i, 0, 0)),
        out_shape=jax.ShapeDtypeStruct((F, Q, d), jnp.float32),
    )(f3, vals, w1p, b1, w2, b2)


def kernel(points, mask, W1, b1, W2, b2):
    pts_m = jnp.where(mask[:, :, None], points, 1e9)
    pts_t = jnp.transpose(pts_m, (0, 2, 1)).reshape(-1)
    feat8 = _sc_encode()(pts_t)
    f3 = feat8.reshape(F, 8, Q)
    d = W2.shape[0]
    w1p = jnp.concatenate([W1, jnp.zeros((3, d), jnp.float32)], axis=0)
    scores = f3[:, 4, :]
    queries = _tc_mlp(f3, scores.reshape(F, Q, 1), w1p,
                      b1.reshape(1, d), W2, b2.reshape(1, d))
    refs = jnp.transpose(f3[:, 0:3, :], (0, 2, 1))
    return queries, refs, scores
